# baseline (speedup 1.0000x reference)
import functools

import jax
import jax.numpy as jnp
from jax import lax
from jax.experimental import pallas as pl
from jax.experimental.pallas import tpu as pltpu
from jax.experimental.pallas import tpu_sc as plsc

B = 16384
D = 32
NC = 2
NS = 16
NW = NC * NS
BPW = B // NW
CHUNK = 128
NCHUNK = BPW // CHUNK
GROUPS = BPW // 16


def _body(u_id_hbm, i_id_hbm, user_emb_hbm, user_bias_hbm, item_emb_hbm,
          item_bias_hbm, mean_hbm, out_hbm,
          uid_v, iid_v, urow_v, irow_v, ub_v, ib_v, out_v, mean_v, sem):
  wid = lax.axis_index("s") * NC + lax.axis_index("c")
  base = wid * BPW

  pltpu.sync_copy(u_id_hbm.at[pl.ds(base, BPW)], uid_v)
  pltpu.sync_copy(i_id_hbm.at[pl.ds(base, BPW)], iid_v)
  pltpu.sync_copy(mean_hbm, mean_v)

  copies = []
  for c in range(NCHUNK):
    s = pl.ds(c * CHUNK, CHUNK)
    copies.append(pltpu.make_async_copy(
        user_emb_hbm.at[uid_v.at[s]], urow_v.at[s, :], sem))
    copies.append(pltpu.make_async_copy(
        item_emb_hbm.at[iid_v.at[s]], irow_v.at[s, :], sem))
    copies.append(pltpu.make_async_copy(
        user_bias_hbm.at[uid_v.at[s]], ub_v.at[s], sem))
    copies.append(pltpu.make_async_copy(
        item_bias_hbm.at[iid_v.at[s]], ib_v.at[s], sem))
  for cp in copies:
    cp.start()
  for cp in copies:
    cp.wait()

  lanes = lax.iota(jnp.int32, 16)
  zeros = jnp.zeros((16,), jnp.int32)
  mean16 = mean_v[...]

  def group_body(g, carry):
    rows = g * 16 + lanes
    acc = mean16
    acc = acc + ub_v[pl.ds(g * 16, 16)]
    acc = acc + ib_v[pl.ds(g * 16, 16)]
    for d in range(D):
      col = jnp.full((16,), d, jnp.int32)
      u = plsc.load_gather(urow_v, [rows, col])
      i = plsc.load_gather(irow_v, [rows, col])
      acc = acc + u * i
    out_v[pl.ds(g * 16, 16)] = acc
    return carry

  lax.fori_loop(0, GROUPS, group_body, 0)

  pltpu.sync_copy(out_v, out_hbm.at[pl.ds(base, BPW)])


@jax.jit
def kernel(u_id, i_id, user_emb, user_bias, item_emb, item_bias, mean):
  mesh = plsc.VectorSubcoreMesh(core_axis_name="c", subcore_axis_name="s")
  f = pl.kernel(
      _body,
      out_type=jax.ShapeDtypeStruct((B,), jnp.float32),
      mesh=mesh,
      scratch_types=[
          pltpu.VMEM((BPW,), jnp.int32),
          pltpu.VMEM((BPW,), jnp.int32),
          pltpu.VMEM((BPW, D), jnp.float32),
          pltpu.VMEM((BPW, D), jnp.float32),
          pltpu.VMEM((BPW,), jnp.float32),
          pltpu.VMEM((BPW,), jnp.float32),
          pltpu.VMEM((BPW,), jnp.float32),
          pltpu.VMEM((16,), jnp.float32),
          pltpu.SemaphoreType.DMA,
      ],
      compiler_params=pltpu.CompilerParams(
          needs_layout_passes=False, use_tc_tiling_on_sc=False),
  )
  mean16 = jnp.broadcast_to(mean.astype(jnp.float32), (16,))
  return f(u_id, i_id, user_emb, user_bias.reshape(-1), item_emb,
           item_bias.reshape(-1), mean16)
